# baseline (speedup 1.0000x reference)
import functools

import jax
import jax.numpy as jnp
from jax.experimental import pallas as pl
from jax.experimental.pallas import tpu as pltpu

_QL = 4
_TK = 3
_NCH = 10
_BG = 8


def _tc_body(mqt, pqt, tae, ego, ptm, pem, ete,
             o1, o2, o6, o3, o4, o5, o7, o8):
    b2 = pl.program_id(0)
    nc = pl.program_id(1)

    mqv = mqt[...]
    taev = tae[...]
    cn = mqv.shape[0]
    for i in range(_BG):
        o1[i] = jnp.broadcast_to(mqv[:, i, None, :], (cn, _QL, mqv.shape[2]))
        for t in range(_TK):
            o6[t, :, i, :] = taev[i, :, t, :]

    @pl.when(nc == 0)
    def _plan():
        pqv = pqt[...]
        for i in range(_BG):
            o2[i] = jnp.broadcast_to(
                pqv[:, i, None, :], (pqv.shape[0], _QL, pqv.shape[2]))

    @pl.when(jnp.logical_and(nc == 0, b2 == 0))
    def _smalls():
        ego_v = ego[...]
        for q in range(_QL):
            o3[:, q] = ego_v
        o4[...] = jnp.zeros(o4.shape, jnp.int32)

        ptm_v = ptm[...]
        for t in range(_TK):
            o5[t] = ((ptm_v >> (8 * t)) & 1).astype(jnp.int8)

        pem_v = pem[...]
        b0 = (pem_v >> 0) & 1
        b1 = (pem_v >> 8) & 1
        b2_ = (pem_v >> 16) & 1
        for t, bt in enumerate((b0, b1, b2_)):
            o7[:, t] = bt.astype(jnp.int8)

        all_true = (b0 + b1 + b2_) == 3
        ff = jnp.where(b0 == 0, 0, jnp.where(b1 == 0, 1, 2))
        pe0 = ete[:, 0]
        pe1 = ete[:, 1]
        pe2 = ete[:, 2]
        last = ete[:, 3]
        tmp = jnp.where(ff == 0, pe0, jnp.where(ff == 1, pe1, pe2))
        for t, pet in enumerate((pe0, pe1, pe2)):
            val = jnp.where(all_true, last, jnp.where(t < ff, tmp, pet))
            o8[:, t, 0] = val


def kernel(motion_query, plan_query, ego_status_feature, mask,
           temp_anchor_embed_forstate, temp_mask_forstate,
           ego_temp_anchor_embed_forstate, ego_temp_mask_forstate):
    del mask
    B, N, D = motion_query.shape
    P = plan_query.shape[1]
    CN = N // _NCH
    NB2 = B // _BG
    sq = pl.squeezed

    mqt = jnp.swapaxes(motion_query, 0, 1)
    pqt = jnp.swapaxes(plan_query, 0, 1)

    ptm = jax.lax.bitcast_convert_type(
        temp_mask_forstate.astype(jnp.uint8), jnp.int32)
    pem = jax.lax.bitcast_convert_type(
        ego_temp_mask_forstate.astype(jnp.uint8), jnp.int32)
    ete = ego_temp_anchor_embed_forstate.reshape(B, _QL, D)

    o1, o2, o6, o3, o4, o5, o7, o8 = pl.pallas_call(
        _tc_body,
        grid=(NB2, _NCH),
        in_specs=[
            pl.BlockSpec((CN, _BG, D), lambda b2, nc: (nc, b2, 0)),
            pl.BlockSpec((P, _BG, D), lambda b2, nc: (0, b2, 0)),
            pl.BlockSpec((_BG, CN, _QL, D),
                         lambda b2, nc: (b2, nc, 0, 0)),
            pl.BlockSpec((B, 1, D), lambda b2, nc: (0, 0, 0)),
            pl.BlockSpec((B, N), lambda b2, nc: (0, 0)),
            pl.BlockSpec((B, 1), lambda b2, nc: (0, 0)),
            pl.BlockSpec((B, _QL, D), lambda b2, nc: (0, 0, 0)),
        ],
        out_specs=[
            pl.BlockSpec((_BG, CN, _QL, D),
                         lambda b2, nc: (b2, nc, 0, 0)),
            pl.BlockSpec((_BG, P, _QL, D),
                         lambda b2, nc: (b2, 0, 0, 0)),
            pl.BlockSpec((_TK, CN, _BG, D),
                         lambda b2, nc: (0, nc, b2, 0)),
            pl.BlockSpec((B, _QL, 1, D), lambda b2, nc: (0, 0, 0, 0)),
            pl.BlockSpec((_QL, B), lambda b2, nc: (0, 0)),
            pl.BlockSpec((_TK, B, N), lambda b2, nc: (0, 0, 0)),
            pl.BlockSpec((B, _TK, 1), lambda b2, nc: (0, 0, 0)),
            pl.BlockSpec((B, _TK, 1, D), lambda b2, nc: (0, 0, 0, 0)),
        ],
        out_shape=[
            jax.ShapeDtypeStruct((B, N, _QL, D), jnp.float32),
            jax.ShapeDtypeStruct((B, P, _QL, D), jnp.float32),
            jax.ShapeDtypeStruct((_TK, N, B, D), jnp.float32),
            jax.ShapeDtypeStruct((B, _QL, 1, D), jnp.float32),
            jax.ShapeDtypeStruct((_QL, B), jnp.int32),
            jax.ShapeDtypeStruct((_TK, B, N), jnp.int8),
            jax.ShapeDtypeStruct((B, _TK, 1), jnp.int8),
            jax.ShapeDtypeStruct((B, _TK, 1, D), jnp.float32),
        ],
    )(mqt, pqt, temp_anchor_embed_forstate, ego_status_feature, ptm, pem, ete)

    out1 = jnp.swapaxes(o1, 1, 2)
    out2 = jnp.swapaxes(o2, 1, 2)
    out6 = jnp.transpose(o6, (2, 0, 1, 3))
    out4 = jnp.swapaxes(o4, 0, 1)
    out5 = jnp.swapaxes(o5, 0, 1)
    return (out1, out2, o3, out4,
            out5.astype(bool), out6, o7.astype(bool), o8)


# speedup vs baseline: 1.0313x; 1.0313x over previous
import functools

import jax
import jax.numpy as jnp
from jax.experimental import pallas as pl
from jax.experimental.pallas import tpu as pltpu

_QL = 4
_TK = 3
_NCH = 4
_BG = 8


def _tc_body(mqt, pqt, tae, ego, ptm, pem, ete,
             o1, o2, o6, o3, o4, o5, o7, o8):
    b2 = pl.program_id(0)
    nc = pl.program_id(1)

    mqv = mqt[...]
    taev = tae[...]
    cn = mqv.shape[0]
    for i in range(_BG):
        o1[i] = jnp.broadcast_to(mqv[:, i, None, :], (cn, _QL, mqv.shape[2]))
        for t in range(_TK):
            o6[t, :, i, :] = taev[i, :, t, :]

    @pl.when(nc == 0)
    def _plan():
        pqv = pqt[...]
        for i in range(_BG):
            o2[i] = jnp.broadcast_to(
                pqv[:, i, None, :], (pqv.shape[0], _QL, pqv.shape[2]))

    @pl.when(jnp.logical_and(nc == 0, b2 == 0))
    def _smalls():
        ego_v = ego[...]
        for q in range(_QL):
            o3[:, q] = ego_v
        o4[...] = jnp.zeros(o4.shape, jnp.int32)

        ptm_v = ptm[...]
        for t in range(_TK):
            o5[t] = ((ptm_v >> (8 * t)) & 1).astype(jnp.int8)

        pem_v = pem[...]
        b0 = (pem_v >> 0) & 1
        b1 = (pem_v >> 8) & 1
        b2_ = (pem_v >> 16) & 1
        for t, bt in enumerate((b0, b1, b2_)):
            o7[:, t] = bt.astype(jnp.int8)

        all_true = (b0 + b1 + b2_) == 3
        ff = jnp.where(b0 == 0, 0, jnp.where(b1 == 0, 1, 2))
        pe0 = ete[:, 0]
        pe1 = ete[:, 1]
        pe2 = ete[:, 2]
        last = ete[:, 3]
        tmp = jnp.where(ff == 0, pe0, jnp.where(ff == 1, pe1, pe2))
        for t, pet in enumerate((pe0, pe1, pe2)):
            val = jnp.where(all_true, last, jnp.where(t < ff, tmp, pet))
            o8[:, t, 0] = val


def kernel(motion_query, plan_query, ego_status_feature, mask,
           temp_anchor_embed_forstate, temp_mask_forstate,
           ego_temp_anchor_embed_forstate, ego_temp_mask_forstate):
    del mask
    B, N, D = motion_query.shape
    P = plan_query.shape[1]
    CN = N // _NCH
    NB2 = B // _BG
    sq = pl.squeezed

    mqt = jnp.swapaxes(motion_query, 0, 1)
    pqt = jnp.swapaxes(plan_query, 0, 1)

    ptm = jax.lax.bitcast_convert_type(
        temp_mask_forstate.astype(jnp.uint8), jnp.int32)
    pem = jax.lax.bitcast_convert_type(
        ego_temp_mask_forstate.astype(jnp.uint8), jnp.int32)
    ete = ego_temp_anchor_embed_forstate.reshape(B, _QL, D)

    o1, o2, o6, o3, o4, o5, o7, o8 = pl.pallas_call(
        _tc_body,
        grid=(NB2, _NCH),
        in_specs=[
            pl.BlockSpec((CN, _BG, D), lambda b2, nc: (nc, b2, 0)),
            pl.BlockSpec((P, _BG, D), lambda b2, nc: (0, b2, 0)),
            pl.BlockSpec((_BG, CN, _QL, D),
                         lambda b2, nc: (b2, nc, 0, 0)),
            pl.BlockSpec((B, 1, D), lambda b2, nc: (0, 0, 0)),
            pl.BlockSpec((B, N), lambda b2, nc: (0, 0)),
            pl.BlockSpec((B, 1), lambda b2, nc: (0, 0)),
            pl.BlockSpec((B, _QL, D), lambda b2, nc: (0, 0, 0)),
        ],
        out_specs=[
            pl.BlockSpec((_BG, CN, _QL, D),
                         lambda b2, nc: (b2, nc, 0, 0)),
            pl.BlockSpec((_BG, P, _QL, D),
                         lambda b2, nc: (b2, 0, 0, 0)),
            pl.BlockSpec((_TK, CN, _BG, D),
                         lambda b2, nc: (0, nc, b2, 0)),
            pl.BlockSpec((B, _QL, 1, D), lambda b2, nc: (0, 0, 0, 0)),
            pl.BlockSpec((_QL, B), lambda b2, nc: (0, 0)),
            pl.BlockSpec((_TK, B, N), lambda b2, nc: (0, 0, 0)),
            pl.BlockSpec((B, _TK, 1), lambda b2, nc: (0, 0, 0)),
            pl.BlockSpec((B, _TK, 1, D), lambda b2, nc: (0, 0, 0, 0)),
        ],
        out_shape=[
            jax.ShapeDtypeStruct((B, N, _QL, D), jnp.float32),
            jax.ShapeDtypeStruct((B, P, _QL, D), jnp.float32),
            jax.ShapeDtypeStruct((_TK, N, B, D), jnp.float32),
            jax.ShapeDtypeStruct((B, _QL, 1, D), jnp.float32),
            jax.ShapeDtypeStruct((_QL, B), jnp.int32),
            jax.ShapeDtypeStruct((_TK, B, N), jnp.int8),
            jax.ShapeDtypeStruct((B, _TK, 1), jnp.int8),
            jax.ShapeDtypeStruct((B, _TK, 1, D), jnp.float32),
        ],
    )(mqt, pqt, temp_anchor_embed_forstate, ego_status_feature, ptm, pem, ete)

    out1 = jnp.swapaxes(o1, 1, 2)
    out2 = jnp.swapaxes(o2, 1, 2)
    out6 = jnp.transpose(o6, (2, 0, 1, 3))
    out4 = jnp.swapaxes(o4, 0, 1)
    out5 = jnp.swapaxes(o5, 0, 1)
    return (out1, out2, o3, out4,
            out5.astype(bool), out6, o7.astype(bool), o8)


# speedup vs baseline: 1.0349x; 1.0035x over previous
import functools

import jax
import jax.numpy as jnp
from jax.experimental import pallas as pl
from jax.experimental.pallas import tpu as pltpu

_QL = 4
_TK = 3
_NCH = 5
_BG = 8


def _tc_body(mqt, pqt, tae, ego, ptm, pem, ete,
             o1, o2, o6, o3, o4, o5, o7, o8):
    b2 = pl.program_id(0)
    nc = pl.program_id(1)

    mqv = mqt[...]
    taev = tae[...]
    cn = mqv.shape[0]
    for i in range(_BG):
        o1[i] = jnp.broadcast_to(mqv[:, i, None, :], (cn, _QL, mqv.shape[2]))
        for t in range(_TK):
            o6[t, :, i, :] = taev[i, :, t, :]

    @pl.when(nc == 0)
    def _plan():
        pqv = pqt[...]
        for i in range(_BG):
            o2[i] = jnp.broadcast_to(
                pqv[:, i, None, :], (pqv.shape[0], _QL, pqv.shape[2]))

    @pl.when(jnp.logical_and(nc == 0, b2 == 0))
    def _smalls():
        ego_v = ego[...]
        for q in range(_QL):
            o3[:, q] = ego_v
        o4[...] = jnp.zeros(o4.shape, jnp.int32)

        ptm_v = ptm[...]
        for t in range(_TK):
            o5[t] = ((ptm_v >> (8 * t)) & 1).astype(jnp.int8)

        pem_v = pem[...]
        b0 = (pem_v >> 0) & 1
        b1 = (pem_v >> 8) & 1
        b2_ = (pem_v >> 16) & 1
        for t, bt in enumerate((b0, b1, b2_)):
            o7[:, t] = bt.astype(jnp.int8)

        all_true = (b0 + b1 + b2_) == 3
        ff = jnp.where(b0 == 0, 0, jnp.where(b1 == 0, 1, 2))
        pe0 = ete[:, 0]
        pe1 = ete[:, 1]
        pe2 = ete[:, 2]
        last = ete[:, 3]
        tmp = jnp.where(ff == 0, pe0, jnp.where(ff == 1, pe1, pe2))
        for t, pet in enumerate((pe0, pe1, pe2)):
            val = jnp.where(all_true, last, jnp.where(t < ff, tmp, pet))
            o8[:, t, 0] = val


def kernel(motion_query, plan_query, ego_status_feature, mask,
           temp_anchor_embed_forstate, temp_mask_forstate,
           ego_temp_anchor_embed_forstate, ego_temp_mask_forstate):
    del mask
    B, N, D = motion_query.shape
    P = plan_query.shape[1]
    CN = N // _NCH
    NB2 = B // _BG

    mqt = jnp.swapaxes(motion_query, 0, 1)
    pqt = jnp.swapaxes(plan_query, 0, 1)

    ptm = jax.lax.bitcast_convert_type(
        temp_mask_forstate.astype(jnp.uint8), jnp.int32)
    pem = jax.lax.bitcast_convert_type(
        ego_temp_mask_forstate.astype(jnp.uint8), jnp.int32)
    ete = ego_temp_anchor_embed_forstate.reshape(B, _QL, D)

    o1, o2, o6, o3, o4, o5, o7, o8 = pl.pallas_call(
        _tc_body,
        grid=(NB2, _NCH),
        in_specs=[
            pl.BlockSpec((CN, _BG, D), lambda b2, nc: (nc, b2, 0)),
            pl.BlockSpec((P, _BG, D), lambda b2, nc: (0, b2, 0)),
            pl.BlockSpec((_BG, CN, _QL, D),
                         lambda b2, nc: (b2, nc, 0, 0)),
            pl.BlockSpec((B, 1, D), lambda b2, nc: (0, 0, 0)),
            pl.BlockSpec((B, N), lambda b2, nc: (0, 0)),
            pl.BlockSpec((B, 1), lambda b2, nc: (0, 0)),
            pl.BlockSpec((B, _QL, D), lambda b2, nc: (0, 0, 0)),
        ],
        out_specs=[
            pl.BlockSpec((_BG, CN, _QL, D),
                         lambda b2, nc: (b2, nc, 0, 0)),
            pl.BlockSpec((_BG, P, _QL, D),
                         lambda b2, nc: (b2, 0, 0, 0)),
            pl.BlockSpec((_TK, CN, _BG, D),
                         lambda b2, nc: (0, nc, b2, 0)),
            pl.BlockSpec((B, _QL, 1, D), lambda b2, nc: (0, 0, 0, 0)),
            pl.BlockSpec((_QL, B), lambda b2, nc: (0, 0)),
            pl.BlockSpec((_TK, B, N), lambda b2, nc: (0, 0, 0)),
            pl.BlockSpec((B, _TK, 1), lambda b2, nc: (0, 0, 0)),
            pl.BlockSpec((B, _TK, 1, D), lambda b2, nc: (0, 0, 0, 0)),
        ],
        out_shape=[
            jax.ShapeDtypeStruct((B, N, _QL, D), jnp.float32),
            jax.ShapeDtypeStruct((B, P, _QL, D), jnp.float32),
            jax.ShapeDtypeStruct((_TK, N, B, D), jnp.float32),
            jax.ShapeDtypeStruct((B, _QL, 1, D), jnp.float32),
            jax.ShapeDtypeStruct((_QL, B), jnp.int32),
            jax.ShapeDtypeStruct((_TK, B, N), jnp.int8),
            jax.ShapeDtypeStruct((B, _TK, 1), jnp.int8),
            jax.ShapeDtypeStruct((B, _TK, 1, D), jnp.float32),
        ],
    )(mqt, pqt, temp_anchor_embed_forstate, ego_status_feature, ptm, pem, ete)

    out1 = jnp.swapaxes(o1, 1, 2)
    out2 = jnp.swapaxes(o2, 1, 2)
    out6 = jnp.transpose(o6, (2, 0, 1, 3))
    out4 = jnp.swapaxes(o4, 0, 1)
    out5 = jnp.swapaxes(o5, 0, 1)
    return (out1, out2, o3, out4,
            out5.astype(bool), out6, o7.astype(bool), o8)


# speedup vs baseline: 1.0440x; 1.0088x over previous
import functools

import jax
import jax.numpy as jnp
from jax.experimental import pallas as pl
from jax.experimental.pallas import tpu as pltpu

_QL = 4
_TK = 3
_NCH = 5
_BG = 8


def _tc_body(mqt, pqt, tae, ego, tms, ets, ete,
             o1, o2, o6, o3, o4, o5, o7, o8):
    b2 = pl.program_id(0)
    nc = pl.program_id(1)

    mqv = mqt[...]
    taev = tae[...]
    cn = mqv.shape[0]
    for i in range(_BG):
        o1[i] = jnp.broadcast_to(mqv[:, i, None, :], (cn, _QL, mqv.shape[2]))
        for t in range(_TK):
            o6[t, :, i, :] = taev[i, :, t, :]

    @pl.when(nc == 0)
    def _plan():
        pqv = pqt[...]
        for i in range(_BG):
            o2[i] = jnp.broadcast_to(
                pqv[:, i, None, :], (pqv.shape[0], _QL, pqv.shape[2]))

    @pl.when(jnp.logical_and(nc == 0, b2 == 0))
    def _smalls():
        ego_v = ego[...]
        for q in range(_QL):
            o3[:, q] = ego_v
        o4[...] = jnp.zeros(o4.shape, jnp.int32)

        tms_v = tms[...]
        for t in range(_TK):
            o5[t] = tms_v[:, t, :]

        ets_v = ets[...].astype(jnp.int32)
        b0 = jnp.reshape(ets_v[:, 0, :], (ets_v.shape[2], 1))
        b1 = jnp.reshape(ets_v[:, 1, :], (ets_v.shape[2], 1))
        b2_ = jnp.reshape(ets_v[:, 2, :], (ets_v.shape[2], 1))
        for t, bt in enumerate((b0, b1, b2_)):
            o7[:, t] = bt.reshape(1, bt.shape[0]).astype(jnp.int8)

        all_true = (b0 + b1 + b2_) == 3
        ff = jnp.where(b0 == 0, 0, jnp.where(b1 == 0, 1, 2))
        pe0 = ete[:, 0]
        pe1 = ete[:, 1]
        pe2 = ete[:, 2]
        last = ete[:, 3]
        tmp = jnp.where(ff == 0, pe0, jnp.where(ff == 1, pe1, pe2))
        for t, pet in enumerate((pe0, pe1, pe2)):
            val = jnp.where(all_true, last, jnp.where(t < ff, tmp, pet))
            o8[:, t, 0] = val


def kernel(motion_query, plan_query, ego_status_feature, mask,
           temp_anchor_embed_forstate, temp_mask_forstate,
           ego_temp_anchor_embed_forstate, ego_temp_mask_forstate):
    del mask
    B, N, D = motion_query.shape
    P = plan_query.shape[1]
    CN = N // _NCH
    NB2 = B // _BG

    mqt = jnp.swapaxes(motion_query, 0, 1)
    pqt = jnp.swapaxes(plan_query, 0, 1)

    tms = jnp.swapaxes(temp_mask_forstate.view(jnp.int8), 1, 2)
    ets = jnp.transpose(
        ego_temp_mask_forstate.view(jnp.int8), (1, 2, 0))
    ete = ego_temp_anchor_embed_forstate.reshape(B, _QL, D)

    o1, o2, o6, o3, o4, o5, o7, o8 = pl.pallas_call(
        _tc_body,
        grid=(NB2, _NCH),
        in_specs=[
            pl.BlockSpec((CN, _BG, D), lambda b2, nc: (nc, b2, 0)),
            pl.BlockSpec((P, _BG, D), lambda b2, nc: (0, b2, 0)),
            pl.BlockSpec((_BG, CN, _QL, D),
                         lambda b2, nc: (b2, nc, 0, 0)),
            pl.BlockSpec((B, 1, D), lambda b2, nc: (0, 0, 0)),
            pl.BlockSpec((B, _QL, N), lambda b2, nc: (0, 0, 0)),
            pl.BlockSpec((1, _QL, B), lambda b2, nc: (0, 0, 0)),
            pl.BlockSpec((B, _QL, D), lambda b2, nc: (0, 0, 0)),
        ],
        out_specs=[
            pl.BlockSpec((_BG, CN, _QL, D),
                         lambda b2, nc: (b2, nc, 0, 0)),
            pl.BlockSpec((_BG, P, _QL, D),
                         lambda b2, nc: (b2, 0, 0, 0)),
            pl.BlockSpec((_TK, CN, _BG, D),
                         lambda b2, nc: (0, nc, b2, 0)),
            pl.BlockSpec((B, _QL, 1, D), lambda b2, nc: (0, 0, 0, 0)),
            pl.BlockSpec((_QL, B), lambda b2, nc: (0, 0)),
            pl.BlockSpec((_TK, B, N), lambda b2, nc: (0, 0, 0)),
            pl.BlockSpec((1, _TK, B), lambda b2, nc: (0, 0, 0)),
            pl.BlockSpec((B, _TK, 1, D), lambda b2, nc: (0, 0, 0, 0)),
        ],
        out_shape=[
            jax.ShapeDtypeStruct((B, N, _QL, D), jnp.float32),
            jax.ShapeDtypeStruct((B, P, _QL, D), jnp.float32),
            jax.ShapeDtypeStruct((_TK, N, B, D), jnp.float32),
            jax.ShapeDtypeStruct((B, _QL, 1, D), jnp.float32),
            jax.ShapeDtypeStruct((_QL, B), jnp.int32),
            jax.ShapeDtypeStruct((_TK, B, N), jnp.int8),
            jax.ShapeDtypeStruct((1, _TK, B), jnp.int8),
            jax.ShapeDtypeStruct((B, _TK, 1, D), jnp.float32),
        ],
    )(mqt, pqt, temp_anchor_embed_forstate, ego_status_feature, tms, ets, ete)

    out1 = jnp.swapaxes(o1, 1, 2)
    out2 = jnp.swapaxes(o2, 1, 2)
    out6 = jnp.transpose(o6, (2, 0, 1, 3))
    out4 = jnp.swapaxes(o4, 0, 1)
    out5 = jnp.swapaxes(o5, 0, 1).view(jnp.bool_)
    out7 = jnp.transpose(o7, (2, 1, 0)).view(jnp.bool_)
    return (out1, out2, o3, out4, out5, out6, out7, o8)


# speedup vs baseline: 1.0458x; 1.0017x over previous
import jax
import jax.numpy as jnp
from jax.experimental import pallas as pl
from jax.experimental.pallas import tpu as pltpu

_QL = 4
_TK = 3
_NCH = 5
_BG = 8


def _tc_body(mqt, pqt, tae, ego, tms, ets, ete,
             o1, o2, o6, o3, o4, o5, o7, o8):
    b2 = pl.program_id(0)
    nc = pl.program_id(1)

    mqv = mqt[...]
    taev = tae[...]
    cn = mqv.shape[0]
    for i in range(_BG):
        o1[i] = jnp.broadcast_to(mqv[:, i, None, :], (cn, _QL, mqv.shape[2]))
        for t in range(_TK):
            o6[t, :, i, :] = taev[i, :, t, :]

    @pl.when(nc == 0)
    def _plan():
        pqv = pqt[...]
        for i in range(_BG):
            o2[i] = jnp.broadcast_to(
                pqv[:, i, None, :], (pqv.shape[0], _QL, pqv.shape[2]))

    @pl.when(jnp.logical_and(nc == 0, b2 == 0))
    def _smalls():
        ego_v = ego[...]
        for q in range(_QL):
            o3[:, q] = ego_v
        o4[...] = jnp.zeros(o4.shape, jnp.int32)

        tms_v = tms[...]
        for t in range(_TK):
            o5[t] = tms_v[:, t, :]

        ets_v = ets[...].astype(jnp.int32)
        b0 = jnp.reshape(ets_v[:, 0, :], (ets_v.shape[2], 1))
        b1 = jnp.reshape(ets_v[:, 1, :], (ets_v.shape[2], 1))
        b2_ = jnp.reshape(ets_v[:, 2, :], (ets_v.shape[2], 1))
        for t, bt in enumerate((b0, b1, b2_)):
            o7[:, t] = bt.reshape(1, bt.shape[0]).astype(jnp.int8)

        all_true = (b0 + b1 + b2_) == 3
        ff = jnp.where(b0 == 0, 0, jnp.where(b1 == 0, 1, 2))
        pe0 = ete[:, 0]
        pe1 = ete[:, 1]
        pe2 = ete[:, 2]
        last = ete[:, 3]
        tmp = jnp.where(ff == 0, pe0, jnp.where(ff == 1, pe1, pe2))
        for t, pet in enumerate((pe0, pe1, pe2)):
            val = jnp.where(all_true, last, jnp.where(t < ff, tmp, pet))
            o8[:, t, 0] = val


def kernel(motion_query, plan_query, ego_status_feature, mask,
           temp_anchor_embed_forstate, temp_mask_forstate,
           ego_temp_anchor_embed_forstate, ego_temp_mask_forstate):
    del mask
    B, N, D = motion_query.shape
    P = plan_query.shape[1]
    CN = N // _NCH
    NB2 = B // _BG

    mqt = jnp.swapaxes(motion_query, 0, 1)
    pqt = jnp.swapaxes(plan_query, 0, 1)

    tms = jnp.swapaxes(temp_mask_forstate.view(jnp.int8), 1, 2)
    ets = jnp.transpose(
        ego_temp_mask_forstate.view(jnp.int8), (1, 2, 0))
    ete = ego_temp_anchor_embed_forstate.reshape(B, _QL, D)

    o1, o2, o6, o3, o4, o5, o7, o8 = pl.pallas_call(
        _tc_body,
        grid=(NB2, _NCH),
        in_specs=[
            pl.BlockSpec((CN, _BG, D), lambda b2, nc: (nc, b2, 0)),
            pl.BlockSpec((P, _BG, D), lambda b2, nc: (0, b2, 0)),
            pl.BlockSpec((_BG, CN, _QL, D),
                         lambda b2, nc: (b2, nc, 0, 0)),
            pl.BlockSpec((B, 1, D), lambda b2, nc: (0, 0, 0)),
            pl.BlockSpec((B, _QL, N), lambda b2, nc: (0, 0, 0)),
            pl.BlockSpec((1, _QL, B), lambda b2, nc: (0, 0, 0)),
            pl.BlockSpec((B, _QL, D), lambda b2, nc: (0, 0, 0)),
        ],
        out_specs=[
            pl.BlockSpec((_BG, CN, _QL, D),
                         lambda b2, nc: (b2, nc, 0, 0)),
            pl.BlockSpec((_BG, P, _QL, D),
                         lambda b2, nc: (b2, 0, 0, 0)),
            pl.BlockSpec((_TK, CN, _BG, D),
                         lambda b2, nc: (0, nc, b2, 0)),
            pl.BlockSpec((B, _QL, 1, D), lambda b2, nc: (0, 0, 0, 0)),
            pl.BlockSpec((_QL, B), lambda b2, nc: (0, 0)),
            pl.BlockSpec((_TK, B, N), lambda b2, nc: (0, 0, 0)),
            pl.BlockSpec((1, _TK, B), lambda b2, nc: (0, 0, 0)),
            pl.BlockSpec((B, _TK, 1, D), lambda b2, nc: (0, 0, 0, 0)),
        ],
        out_shape=[
            jax.ShapeDtypeStruct((B, N, _QL, D), jnp.float32),
            jax.ShapeDtypeStruct((B, P, _QL, D), jnp.float32),
            jax.ShapeDtypeStruct((_TK, N, B, D), jnp.float32),
            jax.ShapeDtypeStruct((B, _QL, 1, D), jnp.float32),
            jax.ShapeDtypeStruct((_QL, B), jnp.int32),
            jax.ShapeDtypeStruct((_TK, B, N), jnp.int8),
            jax.ShapeDtypeStruct((1, _TK, B), jnp.int8),
            jax.ShapeDtypeStruct((B, _TK, 1, D), jnp.float32),
        ],
    )(mqt, pqt, temp_anchor_embed_forstate, ego_status_feature, tms, ets, ete)

    out1 = jnp.swapaxes(o1, 1, 2)
    out2 = jnp.swapaxes(o2, 1, 2)
    out6 = jnp.transpose(o6, (2, 0, 1, 3))
    out4 = jnp.swapaxes(o4, 0, 1)
    out5 = jnp.swapaxes(o5, 0, 1).view(jnp.bool_)
    out7 = jnp.transpose(o7, (2, 1, 0)).view(jnp.bool_)
    return (out1, out2, o3, out4, out5, out6, out7, o8)
